# baseline (speedup 1.0000x reference)
import jax
import jax.numpy as jnp
from jax import lax
from jax.experimental import pallas as pl
from jax.experimental.pallas import tpu as pltpu
from jax.experimental.pallas import tpu_sc as plsc

_W = 7808
_CHUNKS = [_W] * 12 + [6272]
_NCH = len(_CHUNKS)
_TAIL = 100000 - sum(_CHUNKS)
_GROUP = 8


def _uniq_body(aw_ref, ent_ref, u_ref):
    u_ref[...] = jax.lax.dot_general(
        aw_ref[...], ent_ref[...], (((1,), (1,)), ((), ())),
        preferred_element_type=jnp.float32)


_APAD = 56


def _tc_uniq_chunked(attr_weights, ent_emb):
    a, r = attr_weights.shape
    aw_pad = jnp.pad(attr_weights, ((0, _APAD - a), (0, 0)))
    return pl.pallas_call(
        _uniq_body,
        grid=(_NCH,),
        in_specs=[
            pl.BlockSpec((_APAD, r), lambda i: (0, 0)),
            pl.BlockSpec((_W, r), lambda i: (i, 0)),
        ],
        out_specs=pl.BlockSpec((_APAD, _W), lambda i: (i, 0)),
        out_shape=jax.ShapeDtypeStruct((_NCH * _APAD, _W), jnp.float32),
    )(aw_pad, ent_emb)


def _sc_scatter(uniq_chunked, attributes, b, n):
    nc, ns = 2, 16
    nw = nc * ns
    rpw = b // nw
    ngrp = rpw // _GROUP
    mesh = plsc.VectorSubcoreMesh(core_axis_name="c", subcore_axis_name="s")

    def body(uniq_hbm, idx_hbm, out_hbm, idx_v, idx_c, rb0, rb1,
             g0, g1, s0, s1):
        rbs, gsems, ssems = (rb0, rb1), (g0, g1), (s0, s1)
        wid = lax.axis_index("s") * nc + lax.axis_index("c")
        row0 = wid * rpw
        pltpu.sync_copy(idx_hbm.at[pl.ds(row0, rpw)], idx_v)
        lo = idx_v[pl.ds(0, 16)]
        hi = idx_v[pl.ds(16, 16)]

        pending = [None, None]
        step = 0
        for c in range(_NCH):
            w = _CHUNKS[c]
            idx_c[pl.ds(0, 16)] = lo + c * _APAD
            idx_c[pl.ds(16, 16)] = hi + c * _APAD
            col0 = c * _W
            for g in range(ngrp):
                sl = step % 2
                if pending[sl] is not None:
                    pending[sl].wait()
                gcp = pltpu.async_copy(
                    uniq_hbm.at[idx_c.at[pl.ds(g * _GROUP, _GROUP)]],
                    rbs[sl], gsems[sl])
                gcp.wait()
                pending[sl] = pltpu.async_copy(
                    rbs[sl].at[:, pl.ds(0, w)],
                    out_hbm.at[pl.ds(row0 + g * _GROUP, _GROUP),
                               pl.ds(col0, w)],
                    ssems[sl])
                step += 1
        for h in pending:
            if h is not None:
                h.wait()

    k = pl.kernel(
        body,
        out_type=jax.ShapeDtypeStruct((b, n), jnp.float32),
        mesh=mesh,
        scratch_types=[
            pltpu.VMEM((rpw,), jnp.int32),
            pltpu.VMEM((rpw,), jnp.int32),
            pltpu.VMEM((_GROUP, _W), jnp.float32),
            pltpu.VMEM((_GROUP, _W), jnp.float32),
            pltpu.SemaphoreType.DMA,
            pltpu.SemaphoreType.DMA,
            pltpu.SemaphoreType.DMA,
            pltpu.SemaphoreType.DMA,
        ],
    )
    return k(uniq_chunked, attributes)


def _tail_body(attr_ref, aw_ref, ent_ref, _, out_ref):
    attrs = attr_ref[...]
    a = aw_ref.shape[0]
    iota = jax.lax.broadcasted_iota(jnp.int32, (a, attrs.shape[1]), 0)
    onehot = (iota == attrs).astype(jnp.float32)
    gathered = jax.lax.dot_general(
        onehot, aw_ref[...], (((0,), (0,)), ((), ())),
        preferred_element_type=jnp.float32)
    out_ref[...] = jax.lax.dot_general(
        gathered, ent_ref[...], (((1,), (1,)), ((), ())),
        preferred_element_type=jnp.float32)


def kernel(ent_emb, attr_weights, attributes):
    n, r = ent_emb.shape
    b = attributes.shape[0]
    a = attr_weights.shape[0]
    uniq = _tc_uniq_chunked(attr_weights, ent_emb)
    out0 = _sc_scatter(uniq, attributes, b, n)
    attrs2d = attributes.reshape(1, b)
    tail_blk = (n - _TAIL) // 128
    return pl.pallas_call(
        _tail_body,
        grid=(1,),
        in_specs=[
            pl.BlockSpec((1, b), lambda i: (0, 0)),
            pl.BlockSpec((a, r), lambda i: (0, 0)),
            pl.BlockSpec((128, r), lambda i: (tail_blk, 0)),
            pl.BlockSpec(memory_space=pl.ANY),
        ],
        out_specs=pl.BlockSpec((b, 128), lambda i: (0, tail_blk)),
        out_shape=jax.ShapeDtypeStruct((b, n), jnp.float32),
        input_output_aliases={3: 0},
    )(attrs2d, attr_weights, ent_emb, out0)


# speedup vs baseline: 1.4433x; 1.4433x over previous
import jax
import jax.numpy as jnp
from jax.experimental import pallas as pl
from jax.experimental.pallas import tpu as pltpu

_TILE_N = 4096


def _body(attr_ref, aw_ref, ent_ref, out_ref, gathered_ref):
    @pl.when(pl.program_id(0) == 0)
    def _():
        attrs = attr_ref[...]
        a = aw_ref.shape[0]
        iota = jax.lax.broadcasted_iota(jnp.int32, (a, attrs.shape[1]), 0)
        onehot = (iota == attrs).astype(jnp.float32)
        gathered_ref[...] = jax.lax.dot_general(
            onehot, aw_ref[...], (((0,), (0,)), ((), ())),
            preferred_element_type=jnp.float32)

    out_ref[...] = jax.lax.dot_general(
        gathered_ref[...], ent_ref[...], (((1,), (1,)), ((), ())),
        preferred_element_type=jnp.float32)


def kernel(ent_emb, attr_weights, attributes):
    n, r = ent_emb.shape
    b = attributes.shape[0]
    a = attr_weights.shape[0]
    attrs2d = attributes.reshape(1, b)
    return pl.pallas_call(
        _body,
        grid=(pl.cdiv(n, _TILE_N),),
        in_specs=[
            pl.BlockSpec((1, b), lambda i: (0, 0)),
            pl.BlockSpec((a, r), lambda i: (0, 0)),
            pl.BlockSpec((_TILE_N, r), lambda i: (i, 0)),
        ],
        out_specs=pl.BlockSpec((b, _TILE_N), lambda i: (0, i)),
        out_shape=jax.ShapeDtypeStruct((b, n), jnp.float32),
        scratch_shapes=[pltpu.VMEM((b, r), jnp.float32)],
    )(attrs2d, attr_weights, ent_emb)
